# Initial kernel scaffold; baseline (speedup 1.0000x reference)
import jax
import jax.numpy as jnp
from jax.experimental import pallas as pl


def kernel(a_features, a_coords, b_coords, W_conv, W_deconv):
    raise NotImplementedError("write your pallas kernel here")


# speedup vs baseline: 1.5264x; 1.5264x over previous
import functools

import jax
import jax.numpy as jnp
from jax import lax
from jax.experimental import pallas as pl
from jax.experimental.pallas import tpu as pltpu
from jax.experimental.pallas import tpu_sc as plsc

C = 64
GRID = 32
CELLS = GRID ** 3
HALF = CELLS // 2
NSC = 2
NTILE = 16
BATCH = 112
TC_TILE = 512


def _tc_transform_body(a_ref, u_ref, v_ref, w_ref, wt_ref, o_ref):
    a = a_ref[...]
    u = u_ref[0]
    v = v_ref[0]
    w = w_ref[0]
    pid = (u & 1) * 4 + (v & 1) * 2 + (w & 1)
    zero = jnp.zeros_like(a)
    wide = jnp.concatenate(
        [jnp.where(pid == m, a, zero) for m in range(8)], axis=1)
    o_ref[...] = jnp.dot(wide, wt_ref[...], preferred_element_type=jnp.float32)


def _tc_transform(a, u, v, w, wstack):
    nblk = a.shape[0] // TC_TILE
    return pl.pallas_call(
        _tc_transform_body,
        grid=(nblk,),
        in_specs=[
            pl.BlockSpec((TC_TILE, C), lambda i: (i, 0)),
            pl.BlockSpec((1, TC_TILE, 1), lambda i: (i, 0, 0)),
            pl.BlockSpec((1, TC_TILE, 1), lambda i: (i, 0, 0)),
            pl.BlockSpec((1, TC_TILE, 1), lambda i: (i, 0, 0)),
            pl.BlockSpec((8 * C, C), lambda i: (0, 0)),
        ],
        out_specs=pl.BlockSpec((TC_TILE, C), lambda i: (i, 0)),
        out_shape=jax.ShapeDtypeStruct(a.shape, jnp.float32),
    )(a, u, v, w, wstack)


def _cell_16(u, v, w):
    return (u >> 1) * (GRID * GRID) + (v >> 1) * GRID + (w >> 1)


def _make_sc_scatter(npad):
    t_pts = npad // NTILE
    t_rows = t_pts // BATCH
    n_sub = t_pts // (BATCH * 7)
    sub_pts = t_pts // n_sub
    sub_rows = t_rows // n_sub
    z_rows = 1032
    acc_rows = z_rows * NTILE
    mesh = plsc.VectorSubcoreMesh(core_axis_name="c", subcore_axis_name="s")

    @functools.partial(
        pl.kernel,
        out_type=jax.ShapeDtypeStruct((CELLS, C), jnp.float32),
        mesh=mesh,
        scratch_types=[
            pltpu.VMEM((t_rows, BATCH), jnp.int32),
            pltpu.VMEM((t_rows, BATCH), jnp.int32),
            pltpu.VMEM((t_rows, BATCH), jnp.int32),
            pltpu.VMEM((t_rows, BATCH), jnp.int32),
            pltpu.VMEM((sub_pts, C), jnp.float32),
            pltpu.VMEM_SHARED((acc_rows, C), jnp.float32),
            pltpu.SemaphoreType.DMA,
        ],
        compiler_params=pltpu.CompilerParams(use_tc_tiling_on_sc=False),
    )
    def scatter_kernel(h_hbm, u_hbm, v_hbm, w_hbm, zeros_hbm, x_hbm,
                       u_v, v_v, w_v, idx_v, rows_v, acc, sem):
        c = lax.axis_index("c")
        s = lax.axis_index("s")

        pltpu.sync_copy(zeros_hbm, acc.at[pl.ds(s * z_rows, z_rows)])

        pltpu.sync_copy(u_hbm.at[s], u_v)
        pltpu.sync_copy(v_hbm.at[s], v_v)
        pltpu.sync_copy(w_hbm.at[s], w_v)

        base_cell = c * HALF
        trash = HALF + s

        def idx_body(r, carry):
            for g in range(BATCH // 16):
                sl = pl.ds(g * 16, 16)
                cell = _cell_16(u_v[r, sl], v_v[r, sl], w_v[r, sl])
                local = cell - base_cell
                valid = (local >= 0) & (local < HALF)
                idx_v[r, sl] = jnp.where(valid, local, trash)
            return carry

        lax.fori_loop(0, t_rows, idx_body, 0, unroll=False)

        plsc.subcore_barrier()

        for j in range(n_sub):
            pltpu.sync_copy(
                h_hbm.at[pl.ds(s * t_pts + j * sub_pts, sub_pts)], rows_v)
            descs = [
                pltpu.async_copy(
                    rows_v.at[pl.ds(k * BATCH, BATCH)],
                    acc.at[idx_v.at[j * sub_rows + k]],
                    sem,
                    add=True,
                )
                for k in range(sub_rows)
            ]
            for d in descs:
                d.wait()

        plsc.subcore_barrier()

        out_rows = HALF // NTILE
        pltpu.sync_copy(
            acc.at[pl.ds(s * out_rows, out_rows)],
            x_hbm.at[pl.ds(c * HALF + s * out_rows, out_rows)])

    return scatter_kernel


def _make_sc_gather(npad):
    w_pts = npad // (NSC * NTILE)
    w_rows = w_pts // BATCH
    mesh = plsc.VectorSubcoreMesh(core_axis_name="c", subcore_axis_name="s")

    @functools.partial(
        pl.kernel,
        out_type=jax.ShapeDtypeStruct((npad, C), jnp.float32),
        mesh=mesh,
        scratch_types=[
            pltpu.VMEM((w_rows, BATCH), jnp.int32),
            pltpu.VMEM((w_rows, BATCH), jnp.int32),
            pltpu.VMEM((w_rows, BATCH), jnp.int32),
            pltpu.VMEM((w_rows, BATCH), jnp.int32),
            pltpu.VMEM((w_pts, C), jnp.float32),
            pltpu.SemaphoreType.DMA,
        ],
        compiler_params=pltpu.CompilerParams(use_tc_tiling_on_sc=False),
    )
    def gather_kernel(x_hbm, u_hbm, v_hbm, w_hbm, out_hbm,
                      u_v, v_v, w_v, idx_v, rows_v, sem):
        c = lax.axis_index("c")
        s = lax.axis_index("s")
        wid = s * NSC + c

        pltpu.sync_copy(u_hbm.at[wid], u_v)
        pltpu.sync_copy(v_hbm.at[wid], v_v)
        pltpu.sync_copy(w_hbm.at[wid], w_v)

        def idx_body(r, carry):
            for g in range(BATCH // 16):
                sl = pl.ds(g * 16, 16)
                idx_v[r, sl] = _cell_16(u_v[r, sl], v_v[r, sl], w_v[r, sl])
            return carry

        lax.fori_loop(0, w_rows, idx_body, 0, unroll=False)

        descs = [
            pltpu.async_copy(
                x_hbm.at[idx_v.at[k]],
                rows_v.at[pl.ds(k * BATCH, BATCH)],
                sem,
            )
            for k in range(w_rows)
        ]
        for d in descs:
            d.wait()

        pltpu.sync_copy(rows_v, out_hbm.at[pl.ds(wid * w_pts, w_pts)])

    return gather_kernel


def _pad_coords(coords, npad, fill):
    n = coords.shape[0]
    cols = []
    for d in range(3):
        col = coords[:, d].astype(jnp.int32)
        cols.append(jnp.pad(col, (0, npad - n), constant_values=fill))
    return cols


def kernel(a_features, a_coords, b_coords, W_conv, W_deconv):
    n_a = a_features.shape[0]
    n_b = b_coords.shape[0]
    quant = NSC * NTILE * BATCH * 7
    npad_a = -(-n_a // quant) * quant
    npad_b = -(-n_b // quant) * quant

    wc = W_conv.reshape(8 * C, C)
    wd = W_deconv.reshape(8, C, C)[::-1].reshape(8 * C, C)

    ua, va, wa = _pad_coords(a_coords, npad_a, 2048)
    ub, vb, wb = _pad_coords(b_coords, npad_b, 0)

    nblk_a = npad_a // TC_TILE
    nblk_b = npad_b // TC_TILE
    a_pad = jnp.pad(a_features, ((0, npad_a - n_a), (0, 0)))

    h = _tc_transform(
        a_pad,
        ua.reshape(nblk_a, TC_TILE, 1),
        va.reshape(nblk_a, TC_TILE, 1),
        wa.reshape(nblk_a, TC_TILE, 1),
        wc,
    )

    t_rows = npad_a // (NTILE * BATCH)
    zeros = jnp.zeros((1032, C), jnp.float32)
    x = _make_sc_scatter(npad_a)(
        h,
        ua.reshape(NTILE, t_rows, BATCH),
        va.reshape(NTILE, t_rows, BATCH),
        wa.reshape(NTILE, t_rows, BATCH),
        zeros,
    )

    w_rows = npad_b // (NSC * NTILE * BATCH)
    p = _make_sc_gather(npad_b)(
        x,
        ub.reshape(NSC * NTILE, w_rows, BATCH),
        vb.reshape(NSC * NTILE, w_rows, BATCH),
        wb.reshape(NSC * NTILE, w_rows, BATCH),
    )

    out = _tc_transform(
        p,
        ub.reshape(nblk_b, TC_TILE, 1),
        vb.reshape(nblk_b, TC_TILE, 1),
        wb.reshape(nblk_b, TC_TILE, 1),
        wd,
    )
    return out[:n_b]
